# baseline (speedup 1.0000x reference)
import functools

import jax
import jax.numpy as jnp
from jax import lax
from jax.experimental import pallas as pl
from jax.experimental.pallas import tpu as pltpu
from jax.experimental.pallas import tpu_sc as plsc


def _sc_gather(idx2d, table, n_rows, half):
    info = plsc.get_sparse_core_info()
    nc, ns = info.num_cores, info.num_subcores
    nw = nc * ns
    rows_per_w = n_rows // nw
    KC = 20
    CK = KC * 128
    n_chunks = rows_per_w // CK
    assert rows_per_w % CK == 0

    idx_tiles = rows_per_w // 128

    mesh = plsc.VectorSubcoreMesh(core_axis_name="c", subcore_axis_name="s")

    @functools.partial(
        pl.kernel,
        out_type=jax.ShapeDtypeStruct((n_rows, half), jnp.float32),
        mesh=mesh,
        compiler_params=pltpu.CompilerParams(use_tc_tiling_on_sc=False),
        scratch_types=[
            pltpu.VMEM((idx_tiles, 128), jnp.int32),
            pltpu.VMEM((CK, half), jnp.float32),
            pltpu.SemaphoreType.DMA,
        ],
    )
    def gather_kernel(idx_hbm, table_hbm, out_hbm, idx_v, rows_v, sem):
        wid = lax.axis_index("s") * nc + lax.axis_index("c")
        row_base = wid * rows_per_w
        pltpu.sync_copy(idx_hbm.at[pl.ds(wid * idx_tiles, idx_tiles)], idx_v)

        def body(i, carry):
            copies = []
            for j in range(KC):
                copies.append(
                    pltpu.async_copy(
                        table_hbm.at[idx_v.at[i * KC + j]],
                        rows_v.at[pl.ds(j * 128, 128)],
                        sem,
                    )
                )
            for cpy in copies:
                cpy.wait()
            pltpu.sync_copy(rows_v, out_hbm.at[pl.ds(row_base + i * CK, CK)])
            return carry

        lax.fori_loop(0, n_chunks, body, 0)

    return gather_kernel(idx2d, table)


def _tc_table_pack(table_t, sel, vocab, half):
    CB = 512
    OB = CB * half // 128
    n_out = vocab * half // 128
    grid = pl.cdiv(vocab, CB)

    def body(t_ref, sel_ref, out_ref):
        x = t_ref[...]
        g = jax.lax.dot_general(
            sel_ref[...], x, (((0,), (1,)), ((), ())),
            preferred_element_type=jnp.float32,
        )
        out_ref[...] = jnp.concatenate(
            [g[128 * m:128 * (m + 1), :] for m in range(4)], axis=1
        )

    return pl.pallas_call(
        body,
        grid=(grid,),
        in_specs=[
            pl.BlockSpec((half, CB), lambda i: (0, i)),
            pl.BlockSpec((CB, 512), lambda i: (0, 0)),
        ],
        out_specs=pl.BlockSpec((OB, 128), lambda i: (i, 0)),
        out_shape=jax.ShapeDtypeStruct((n_out, 128), jnp.float32),
    )(table_t, sel)


def _tc_fused(emb5, vals_t, w4, u4col, c4col, b, l, hidden):
    BB = 128
    lg = l // 4
    grid = b // BB

    def body(emb_ref, vals_ref, w_ref, u_ref, c_ref, out_ref):
        w = w_ref[...]
        u = u_ref[...]
        c = c_ref[...]
        for g in range(lg):
            e_g = emb_ref[:, g, :]
            ot = jax.lax.dot_general(
                w, e_g, (((0,), (1,)), ((), ())),
                preferred_element_type=jnp.float32,
            )
            vg = vals_ref[pl.ds(4 * g, 4), :]
            vt = jnp.dot(u, vg, preferred_element_type=jnp.float32)
            out_ref[pl.ds(4 * g, 4)] = (ot + vt + c).reshape(4, hidden, BB)

    return pl.pallas_call(
        body,
        grid=(grid,),
        in_specs=[
            pl.BlockSpec((BB, lg, 128), lambda i: (i, 0, 0)),
            pl.BlockSpec((l, BB), lambda i: (0, i)),
            pl.BlockSpec((128, 4 * hidden), lambda i: (0, 0)),
            pl.BlockSpec((4 * hidden, 4), lambda i: (0, 0)),
            pl.BlockSpec((4 * hidden, 1), lambda i: (0, 0)),
        ],
        out_specs=pl.BlockSpec((l, hidden, BB), lambda i: (0, 0, i)),
        out_shape=jax.ShapeDtypeStruct((l, hidden, b), jnp.float32),
    )(emb5, vals_t, w4, u4col, c4col)


def kernel(lab_codes, lab_values, code_table, value_W, value_b, out_W, out_b):
    B, L = lab_codes.shape
    vocab, half = code_table.shape
    hidden = out_W.shape[0]
    n_rows = B * L
    n4 = n_rows // 4

    w1t = out_W[:, :half].T
    w2 = out_W[:, half:]
    u_col = (w2 @ value_W[:, 0]).reshape(hidden, 1)
    c_col = (w2 @ value_b + out_b).reshape(hidden, 1)
    eye4 = jnp.eye(4, dtype=jnp.float32)
    w4 = jnp.kron(eye4, w1t)
    u4col = jnp.kron(eye4, u_col)
    c4col = jnp.tile(c_col, (4, 1))

    p_idx = jnp.arange(512)
    sel = (p_idx[:, None] == (4 * (p_idx[None, :] % 128) + p_idx[None, :] // 128)
           ).astype(jnp.float32)
    packed = _tc_table_pack(code_table.T, sel, vocab, half)
    table2 = packed.reshape(vocab, half)

    idx2d = lab_codes.reshape(n_rows // 128, 128).astype(jnp.int32)
    emb = _sc_gather(idx2d, table2, n_rows, half)
    emb5 = emb.reshape(B, L // 4, 4 * half)

    vals_t = lab_values.T
    out_t = _tc_fused(emb5, vals_t, w4, u4col, c4col, B, L, hidden)
    return out_t.transpose(2, 0, 1)


# speedup vs baseline: 2.6645x; 2.6645x over previous
import functools

import jax
import jax.numpy as jnp
from jax import lax
from jax.experimental import pallas as pl
from jax.experimental.pallas import tpu as pltpu
from jax.experimental.pallas import tpu_sc as plsc


def _sc_gather(idx2d, table, n_rows, half):
    info = plsc.get_sparse_core_info()
    nc, ns = info.num_cores, info.num_subcores
    nw = nc * ns
    rows_per_w = n_rows // nw
    KC = 20
    CK = KC * 128
    n_chunks = rows_per_w // CK
    assert rows_per_w % CK == 0

    idx_tiles = rows_per_w // 128

    mesh = plsc.VectorSubcoreMesh(core_axis_name="c", subcore_axis_name="s")

    @functools.partial(
        pl.kernel,
        out_type=jax.ShapeDtypeStruct((n_rows, half), jnp.float32),
        mesh=mesh,
        compiler_params=pltpu.CompilerParams(use_tc_tiling_on_sc=False),
        scratch_types=[
            pltpu.VMEM((idx_tiles, 128), jnp.int32),
            pltpu.VMEM((CK, half), jnp.float32),
            pltpu.SemaphoreType.DMA,
        ],
    )
    def gather_kernel(idx_hbm, table_hbm, out_hbm, idx_v, rows_v, sem):
        wid = lax.axis_index("s") * nc + lax.axis_index("c")
        row_base = wid * rows_per_w
        pltpu.sync_copy(idx_hbm.at[pl.ds(wid * idx_tiles, idx_tiles)], idx_v)

        def body(i, carry):
            copies = []
            for j in range(KC):
                copies.append(
                    pltpu.async_copy(
                        table_hbm.at[idx_v.at[i * KC + j]],
                        rows_v.at[pl.ds(j * 128, 128)],
                        sem,
                    )
                )
            for cpy in copies:
                cpy.wait()
            pltpu.sync_copy(rows_v, out_hbm.at[pl.ds(row_base + i * CK, CK)])
            return carry

        lax.fori_loop(0, n_chunks, body, 0)

    return gather_kernel(idx2d, table)


def _tc_table_pack(table_t, eye, vocab, half):
    CB = 4096
    OB = CB * half // 128
    vocab_pad = pl.cdiv(vocab, 512) * 512
    n_out = vocab_pad * half // 128
    grid = pl.cdiv(vocab, CB)

    def body(t_ref, eye_ref, out_ref):
        x = t_ref[...]
        ident = eye_ref[...]
        for sch in range(CB // 512):
            xs = x[:, 512 * sch:512 * (sch + 1)]
            y = jnp.transpose(xs, (1, 0))
            z = jnp.concatenate(
                [y[128 * m:128 * (m + 1), :] for m in range(4)], axis=1
            )
            out_ref[pl.ds(128 * sch, 128), :] = z

    return pl.pallas_call(
        body,
        grid=(grid,),
        in_specs=[
            pl.BlockSpec((half, CB), lambda i: (0, i)),
            pl.BlockSpec((half, half), lambda i: (0, 0)),
        ],
        out_specs=pl.BlockSpec((OB, 128), lambda i: (i, 0)),
        out_shape=jax.ShapeDtypeStruct((n_out, 128), jnp.float32),
    )(table_t, eye)


def _tc_fused(emb5, vals_t, w4, u4col, c4col, b, l, hidden):
    BB = 128
    lg = l // 4
    grid = b // BB

    def body(emb_ref, vals_ref, w_ref, u_ref, c_ref, out_ref):
        w = w_ref[...]
        u = u_ref[...]
        c = c_ref[...]
        for g in range(lg):
            e_g = emb_ref[:, g, :]
            ot = jax.lax.dot_general(
                w, e_g, (((0,), (1,)), ((), ())),
                preferred_element_type=jnp.float32,
            )
            vg = vals_ref[pl.ds(4 * g, 4), :]
            vt = jnp.dot(u, vg, preferred_element_type=jnp.float32)
            out_ref[pl.ds(4 * g, 4)] = (ot + vt + c).reshape(4, hidden, BB)

    return pl.pallas_call(
        body,
        grid=(grid,),
        in_specs=[
            pl.BlockSpec((BB, lg, 128), lambda i: (i, 0, 0)),
            pl.BlockSpec((l, BB), lambda i: (0, i)),
            pl.BlockSpec((128, 4 * hidden), lambda i: (0, 0)),
            pl.BlockSpec((4 * hidden, 4), lambda i: (0, 0)),
            pl.BlockSpec((4 * hidden, 1), lambda i: (0, 0)),
        ],
        out_specs=pl.BlockSpec((l, hidden, BB), lambda i: (0, 0, i)),
        out_shape=jax.ShapeDtypeStruct((l, hidden, b), jnp.float32),
    )(emb5, vals_t, w4, u4col, c4col)


def kernel(lab_codes, lab_values, code_table, value_W, value_b, out_W, out_b):
    B, L = lab_codes.shape
    vocab, half = code_table.shape
    hidden = out_W.shape[0]
    n_rows = B * L
    n4 = n_rows // 4

    w1t = out_W[:, :half].T
    w2 = out_W[:, half:]
    u_col = (w2 @ value_W[:, 0]).reshape(hidden, 1)
    c_col = (w2 @ value_b + out_b).reshape(hidden, 1)
    eye4 = jnp.eye(4, dtype=jnp.float32)
    w4 = jnp.kron(eye4, w1t)
    u4col = jnp.kron(eye4, u_col)
    c4col = jnp.tile(c_col, (4, 1))

    eye = jnp.eye(half, dtype=jnp.float32)
    packed = _tc_table_pack(code_table.T, eye, vocab, half)
    table2 = packed.reshape(packed.shape[0] * 128 // half, half)

    idx = lab_codes.reshape(n_rows // 128, 128).astype(jnp.int32)
    idx2d = 512 * (idx // 512) + 4 * (idx % 128) + (idx // 128) % 4
    emb = _sc_gather(idx2d, table2, n_rows, half)
    emb5 = emb.reshape(B, L // 4, 4 * half)

    vals_t = lab_values.T
    out_t = _tc_fused(emb5, vals_t, w4, u4col, c4col, B, L, hidden)
    return out_t.transpose(2, 0, 1)
